# baseline (speedup 1.0000x reference)
import functools

import jax
import jax.numpy as jnp
from jax import lax
from jax.experimental import pallas as pl
from jax.experimental.pallas import tpu as pltpu
from jax.experimental.pallas import tpu_sc as plsc

NC = 2
NS = 16
NW = NC * NS
CHUNK = 128
BLK = 400


def _ln(z, g, b, eps=1e-5):
    mu = jnp.mean(z, axis=1, keepdims=True)
    d = z - mu
    var = jnp.mean(d * d, axis=1, keepdims=True)
    return d * lax.rsqrt(var + eps) * g + b


def _dot_t(a, w):
    return lax.dot_general(a, w, (((1,), (1,)), ((), ())),
                           preferred_element_type=jnp.float32)


def _make_agg(d, np_rows, c_chunks):
    r = np_rows // NS
    nbuf = 2 if d > 64 else 4
    assert c_chunks % nbuf == 0
    mesh = plsc.VectorSubcoreMesh(core_axis_name="c", subcore_axis_name="s")

    @functools.partial(
        pl.kernel,
        mesh=mesh,
        out_type=jax.ShapeDtypeStruct((NC, np_rows, d), jnp.float32),
        scratch_types=(
            [pltpu.VMEM((c_chunks, CHUNK), jnp.int32)] * 2
            + [pltpu.VMEM((CHUNK, d), jnp.float32)] * nbuf
            + [pltpu.VMEM_SHARED((np_rows, d), jnp.float32)]
            + [pltpu.SemaphoreType.DMA] * 2
        ),
        compiler_params=pltpu.CompilerParams(use_tc_tiling_on_sc=False),
    )
    def agg(y_hbm, srcb_hbm, dstb_hbm, zeros_hbm, out_hbm,
            src_v, dst_v, *rest):
        bufs = rest[:nbuf]
        acc = rest[nbuf]
        gsem = rest[nbuf + 1]
        ssem = rest[nbuf + 2]
        cid = lax.axis_index("c")
        sid = lax.axis_index("s")
        wid = sid * NC + cid
        pltpu.sync_copy(srcb_hbm.at[wid], src_v)
        pltpu.sync_copy(dstb_hbm.at[wid], dst_v)
        r0 = sid * r
        pltpu.sync_copy(zeros_hbm.at[pl.ds(r0, r)], acc.at[pl.ds(r0, r)])
        plsc.subcore_barrier()

        last = c_chunks - 1
        for k in range(nbuf):
            pltpu.async_copy(y_hbm.at[src_v.at[k]], bufs[k], gsem)

        def body(i, carry):
            j = nbuf * i
            for k in range(nbuf):
                pltpu.make_async_copy(
                    y_hbm.at[src_v.at[0]], bufs[k], gsem).wait()
            for k in range(nbuf):
                pltpu.async_copy(bufs[k], acc.at[dst_v.at[j + k]], ssem,
                                 add=True)
            for k in range(nbuf):
                pltpu.make_async_copy(
                    bufs[k], acc.at[dst_v.at[0]], ssem).wait()
            for k in range(nbuf):
                nxt = jnp.minimum(j + nbuf + k, last)
                pltpu.async_copy(y_hbm.at[src_v.at[nxt]], bufs[k], gsem)
            return carry

        lax.fori_loop(0, c_chunks // nbuf, body, 0)
        for k in range(nbuf):
            pltpu.make_async_copy(
                y_hbm.at[src_v.at[0]], bufs[k], gsem).wait()
        plsc.subcore_barrier()
        pltpu.sync_copy(acc.at[pl.ds(r0, r)], out_hbm.at[cid, pl.ds(r0, r)])

    return agg


def _make_cnt(np_rows, c_chunks):
    r = np_rows // NS
    mesh = plsc.VectorSubcoreMesh(core_axis_name="c", subcore_axis_name="s")

    @functools.partial(
        pl.kernel,
        mesh=mesh,
        out_type=jax.ShapeDtypeStruct((NC, np_rows, 16), jnp.float32),
        scratch_types=[
            pltpu.VMEM((c_chunks, CHUNK), jnp.int32),
            pltpu.VMEM((CHUNK, 16), jnp.float32),
            pltpu.VMEM_SHARED((np_rows, 16), jnp.float32),
        ],
        compiler_params=pltpu.CompilerParams(use_tc_tiling_on_sc=False),
    )
    def cnt(dstb_hbm, zeros_hbm, ones_hbm, out_hbm, dst_v, ones_v, acc):
        cid = lax.axis_index("c")
        sid = lax.axis_index("s")
        wid = sid * NC + cid
        pltpu.sync_copy(dstb_hbm.at[wid], dst_v)
        pltpu.sync_copy(ones_hbm, ones_v)
        r0 = sid * r
        pltpu.sync_copy(zeros_hbm.at[pl.ds(r0, r)], acc.at[pl.ds(r0, r)])
        plsc.subcore_barrier()

        def body(j, carry):
            pltpu.sync_copy(ones_v, acc.at[dst_v.at[j]], add=True)
            return carry

        lax.fori_loop(0, c_chunks, body, 0)
        plsc.subcore_barrier()
        pltpu.sync_copy(acc.at[pl.ds(r0, r)], out_hbm.at[cid, pl.ds(r0, r)])

    return cnt


def _pre(x, wl1):
    n = x.shape[0]
    grid = (n // BLK,)

    def body(x_ref, w_ref, o_ref):
        o_ref[...] = _dot_t(x_ref[...], w_ref[...])

    return pl.pallas_call(
        body,
        grid=grid,
        in_specs=[
            pl.BlockSpec((BLK, x.shape[1]), lambda i: (i, 0)),
            pl.BlockSpec(wl1.shape, lambda i: (0, 0)),
        ],
        out_specs=pl.BlockSpec((BLK, wl1.shape[0]), lambda i: (i, 0)),
        out_shape=jax.ShapeDtypeStruct((n, wl1.shape[0]), jnp.float32),
    )(x, wl1)


def _mid(h_in, p, cr, wr, b, g, be, wl_next, first):
    n = h_in.shape[0]
    d_out = wr.shape[0]
    np_rows = p.shape[1]
    d_agg = p.shape[2]
    grid = (n // BLK,)

    def body2(*refs):
        i = 0
        h_ref = refs[i]; i += 1
        p_ref = refs[i]; i += 1
        cr_ref = refs[i]; i += 1
        wr_ref = refs[i]; i += 1
        b_ref = refs[i]; i += 1
        g_ref = refs[i]; i += 1
        be_ref = refs[i]; i += 1
        if wl_next is not None:
            wln_ref = refs[i]; i += 1
        h_out = refs[i]; i += 1
        if wl_next is not None:
            y_out = refs[i]; i += 1
        if first:
            rinv_out = refs[i]; i += 1

        s = p_ref[0] + p_ref[1]
        if first:
            cnt = cr_ref[0, :, 0:1] + cr_ref[1, :, 0:1]
            ri = 1.0 / jnp.maximum(cnt, 1.0)
        else:
            ri = cr_ref[:, 0:1]
        mean = s * ri
        z = mean + b_ref[...] + _dot_t(h_ref[...], wr_ref[...])
        h = _ln(jnp.maximum(z, 0.0), g_ref[...], be_ref[...])
        h_out[...] = h
        if wl_next is not None:
            y_out[...] = _dot_t(h, wln_ref[...])
        if first:
            rinv_out[...] = jnp.broadcast_to(ri, (BLK, 16))

    in_specs = [
        pl.BlockSpec((BLK, h_in.shape[1]), lambda i: (i, 0)),
        pl.BlockSpec((NC, BLK, d_agg), lambda i: (0, i, 0)),
    ]
    operands = [h_in, p]
    if first:
        in_specs.append(pl.BlockSpec((NC, BLK, 16), lambda i: (0, i, 0)))
    else:
        in_specs.append(pl.BlockSpec((BLK, 16), lambda i: (i, 0)))
    operands.append(cr)
    in_specs += [
        pl.BlockSpec(wr.shape, lambda i: (0, 0)),
        pl.BlockSpec((1, d_out), lambda i: (0, 0)),
        pl.BlockSpec((1, d_out), lambda i: (0, 0)),
        pl.BlockSpec((1, d_out), lambda i: (0, 0)),
    ]
    operands += [wr, b.reshape(1, -1), g.reshape(1, -1), be.reshape(1, -1)]
    if wl_next is not None:
        in_specs.append(pl.BlockSpec(wl_next.shape, lambda i: (0, 0)))
        operands.append(wl_next)

    out_specs = [pl.BlockSpec((BLK, d_out), lambda i: (i, 0))]
    out_shape = [jax.ShapeDtypeStruct((n, d_out), jnp.float32)]
    if wl_next is not None:
        out_specs.append(pl.BlockSpec((BLK, wl_next.shape[0]), lambda i: (i, 0)))
        out_shape.append(jax.ShapeDtypeStruct((n, wl_next.shape[0]), jnp.float32))
    if first:
        out_specs.append(pl.BlockSpec((BLK, 16), lambda i: (i, 0)))
        out_shape.append(jax.ShapeDtypeStruct((n, 16), jnp.float32))

    return pl.pallas_call(
        body2,
        grid=grid,
        in_specs=in_specs,
        out_specs=out_specs,
        out_shape=out_shape,
    )(*operands)


def kernel(x, edge_index, Wl1, Wr1, b1, g1, be1,
           Wl2, Wr2, b2, g2, be2, Wl3, Wr3, b3, g3, be3):
    n = x.shape[0]
    e = edge_index.shape[1]
    c_chunks = 8 * -(-e // (NW * CHUNK * 8))
    e_pad = NW * c_chunks * CHUNK
    r = -(-(n + 1) // (NS * 8)) * 8
    np_rows = NS * r

    ei = edge_index.astype(jnp.int32)
    src = jnp.concatenate([ei[0], jnp.zeros((e_pad - e,), jnp.int32)])
    dst = jnp.concatenate([ei[1], jnp.full((e_pad - e,), n, jnp.int32)])
    srcb = src.reshape(NW, c_chunks, CHUNK)
    dstb = dst.reshape(NW, c_chunks, CHUNK)

    d1 = Wl1.shape[0]
    d23 = Wl2.shape[0]
    z1 = jnp.zeros((np_rows, d1), jnp.float32)
    z23 = jnp.zeros((np_rows, d23), jnp.float32)
    z16 = jnp.zeros((np_rows, 16), jnp.float32)
    ones16 = jnp.ones((CHUNK, 16), jnp.float32)

    agg1 = _make_agg(d1, np_rows, c_chunks)
    agg23 = _make_agg(d23, np_rows, c_chunks)
    cntk = _make_cnt(np_rows, c_chunks)

    cntp = cntk(dstb, z16, ones16)
    y1 = _pre(x, Wl1)
    p1 = agg1(y1, srcb, dstb, z1)
    h1, y2, rinv = _mid(x, p1, cntp, Wr1, b1, g1, be1, Wl2, True)
    p2 = agg23(y2, srcb, dstb, z23)
    h2, y3 = _mid(h1, p2, rinv, Wr2, b2, g2, be2, Wl3, False)
    p3 = agg23(y3, srcb, dstb, z23)
    (h3,) = _mid(h2, p3, rinv, Wr3, b3, g3, be3, None, False)
    return h3


# speedup vs baseline: 1.0586x; 1.0586x over previous
import functools

import jax
import jax.numpy as jnp
from jax import lax
from jax.experimental import pallas as pl
from jax.experimental.pallas import tpu as pltpu
from jax.experimental.pallas import tpu_sc as plsc

NC = 2
NS = 16
NW = NC * NS
CHUNK = 128
BLK = 400


def _ln(z, g, b, eps=1e-5):
    mu = jnp.mean(z, axis=1, keepdims=True)
    d = z - mu
    var = jnp.mean(d * d, axis=1, keepdims=True)
    return d * lax.rsqrt(var + eps) * g + b


def _dot_t(a, w):
    return lax.dot_general(a, w, (((1,), (1,)), ((), ())),
                           preferred_element_type=jnp.float32)


def _make_agg(d, np_rows, c_chunks):
    r = np_rows // NS
    nbuf = 2
    assert c_chunks % nbuf == 0
    mesh = plsc.VectorSubcoreMesh(core_axis_name="c", subcore_axis_name="s")

    @functools.partial(
        pl.kernel,
        mesh=mesh,
        out_type=jax.ShapeDtypeStruct((NC, np_rows, d), jnp.float32),
        scratch_types=(
            [pltpu.VMEM((c_chunks, CHUNK), jnp.int32)] * 2
            + [pltpu.VMEM((CHUNK, d), jnp.float32)] * nbuf
            + [pltpu.VMEM_SHARED((np_rows, d), jnp.float32)]
            + [pltpu.SemaphoreType.DMA] * nbuf
        ),
        compiler_params=pltpu.CompilerParams(use_tc_tiling_on_sc=False),
    )
    def agg(y_hbm, srcb_hbm, dstb_hbm, zeros_hbm, out_hbm,
            src_v, dst_v, *rest):
        bufs = rest[:nbuf]
        acc = rest[nbuf]
        sems = rest[nbuf + 1:]
        cid = lax.axis_index("c")
        sid = lax.axis_index("s")
        wid = sid * NC + cid
        pltpu.sync_copy(srcb_hbm.at[wid], src_v)
        pltpu.sync_copy(dstb_hbm.at[wid], dst_v)
        r0 = sid * r
        pltpu.sync_copy(zeros_hbm.at[pl.ds(r0, r)], acc.at[pl.ds(r0, r)])
        plsc.subcore_barrier()

        last = c_chunks - 1
        for k in range(nbuf):
            pltpu.async_copy(y_hbm.at[src_v.at[k]], bufs[k], sems[k])

        def body(i, carry):
            j = nbuf * i
            for k in range(nbuf):
                pltpu.make_async_copy(
                    y_hbm.at[src_v.at[0]], bufs[k], sems[k]).wait()
                pltpu.async_copy(bufs[k], acc.at[dst_v.at[j + k]], sems[k],
                                 add=True)
            for k in range(nbuf):
                pltpu.make_async_copy(
                    bufs[k], acc.at[dst_v.at[0]], sems[k]).wait()
                nxt = jnp.minimum(j + nbuf + k, last)
                pltpu.async_copy(y_hbm.at[src_v.at[nxt]], bufs[k], sems[k])
            return carry

        lax.fori_loop(0, c_chunks // nbuf, body, 0)
        for k in range(nbuf):
            pltpu.make_async_copy(
                y_hbm.at[src_v.at[0]], bufs[k], sems[k]).wait()
        plsc.subcore_barrier()
        pltpu.sync_copy(acc.at[pl.ds(r0, r)], out_hbm.at[cid, pl.ds(r0, r)])

    return agg


def _make_cnt(np_rows, c_chunks):
    r = np_rows // NS
    mesh = plsc.VectorSubcoreMesh(core_axis_name="c", subcore_axis_name="s")

    @functools.partial(
        pl.kernel,
        mesh=mesh,
        out_type=jax.ShapeDtypeStruct((NC, np_rows, 16), jnp.float32),
        scratch_types=[
            pltpu.VMEM((c_chunks, CHUNK), jnp.int32),
            pltpu.VMEM((CHUNK, 16), jnp.float32),
            pltpu.VMEM_SHARED((np_rows, 16), jnp.float32),
        ],
        compiler_params=pltpu.CompilerParams(use_tc_tiling_on_sc=False),
    )
    def cnt(dstb_hbm, zeros_hbm, ones_hbm, out_hbm, dst_v, ones_v, acc):
        cid = lax.axis_index("c")
        sid = lax.axis_index("s")
        wid = sid * NC + cid
        pltpu.sync_copy(dstb_hbm.at[wid], dst_v)
        pltpu.sync_copy(ones_hbm, ones_v)
        r0 = sid * r
        pltpu.sync_copy(zeros_hbm.at[pl.ds(r0, r)], acc.at[pl.ds(r0, r)])
        plsc.subcore_barrier()

        def body(j, carry):
            pltpu.sync_copy(ones_v, acc.at[dst_v.at[j]], add=True)
            return carry

        lax.fori_loop(0, c_chunks, body, 0)
        plsc.subcore_barrier()
        pltpu.sync_copy(acc.at[pl.ds(r0, r)], out_hbm.at[cid, pl.ds(r0, r)])

    return cnt


def _pre(x, wl1):
    n = x.shape[0]
    grid = (n // BLK,)

    def body(x_ref, w_ref, o_ref):
        o_ref[...] = _dot_t(x_ref[...], w_ref[...])

    return pl.pallas_call(
        body,
        grid=grid,
        in_specs=[
            pl.BlockSpec((BLK, x.shape[1]), lambda i: (i, 0)),
            pl.BlockSpec(wl1.shape, lambda i: (0, 0)),
        ],
        out_specs=pl.BlockSpec((BLK, wl1.shape[0]), lambda i: (i, 0)),
        out_shape=jax.ShapeDtypeStruct((n, wl1.shape[0]), jnp.float32),
    )(x, wl1)


def _mid(h_in, p, cr, wr, b, g, be, wl_next, first):
    n = h_in.shape[0]
    d_out = wr.shape[0]
    np_rows = p.shape[1]
    d_agg = p.shape[2]
    grid = (n // BLK,)

    def body2(*refs):
        i = 0
        h_ref = refs[i]; i += 1
        p_ref = refs[i]; i += 1
        cr_ref = refs[i]; i += 1
        wr_ref = refs[i]; i += 1
        b_ref = refs[i]; i += 1
        g_ref = refs[i]; i += 1
        be_ref = refs[i]; i += 1
        if wl_next is not None:
            wln_ref = refs[i]; i += 1
        h_out = refs[i]; i += 1
        if wl_next is not None:
            y_out = refs[i]; i += 1
        if first:
            rinv_out = refs[i]; i += 1

        s = p_ref[0] + p_ref[1]
        if first:
            cnt = cr_ref[0, :, 0:1] + cr_ref[1, :, 0:1]
            ri = 1.0 / jnp.maximum(cnt, 1.0)
        else:
            ri = cr_ref[:, 0:1]
        mean = s * ri
        z = mean + b_ref[...] + _dot_t(h_ref[...], wr_ref[...])
        h = _ln(jnp.maximum(z, 0.0), g_ref[...], be_ref[...])
        h_out[...] = h
        if wl_next is not None:
            y_out[...] = _dot_t(h, wln_ref[...])
        if first:
            rinv_out[...] = jnp.broadcast_to(ri, (BLK, 16))

    in_specs = [
        pl.BlockSpec((BLK, h_in.shape[1]), lambda i: (i, 0)),
        pl.BlockSpec((NC, BLK, d_agg), lambda i: (0, i, 0)),
    ]
    operands = [h_in, p]
    if first:
        in_specs.append(pl.BlockSpec((NC, BLK, 16), lambda i: (0, i, 0)))
    else:
        in_specs.append(pl.BlockSpec((BLK, 16), lambda i: (i, 0)))
    operands.append(cr)
    in_specs += [
        pl.BlockSpec(wr.shape, lambda i: (0, 0)),
        pl.BlockSpec((1, d_out), lambda i: (0, 0)),
        pl.BlockSpec((1, d_out), lambda i: (0, 0)),
        pl.BlockSpec((1, d_out), lambda i: (0, 0)),
    ]
    operands += [wr, b.reshape(1, -1), g.reshape(1, -1), be.reshape(1, -1)]
    if wl_next is not None:
        in_specs.append(pl.BlockSpec(wl_next.shape, lambda i: (0, 0)))
        operands.append(wl_next)

    out_specs = [pl.BlockSpec((BLK, d_out), lambda i: (i, 0))]
    out_shape = [jax.ShapeDtypeStruct((n, d_out), jnp.float32)]
    if wl_next is not None:
        out_specs.append(pl.BlockSpec((BLK, wl_next.shape[0]), lambda i: (i, 0)))
        out_shape.append(jax.ShapeDtypeStruct((n, wl_next.shape[0]), jnp.float32))
    if first:
        out_specs.append(pl.BlockSpec((BLK, 16), lambda i: (i, 0)))
        out_shape.append(jax.ShapeDtypeStruct((n, 16), jnp.float32))

    return pl.pallas_call(
        body2,
        grid=grid,
        in_specs=in_specs,
        out_specs=out_specs,
        out_shape=out_shape,
    )(*operands)


def kernel(x, edge_index, Wl1, Wr1, b1, g1, be1,
           Wl2, Wr2, b2, g2, be2, Wl3, Wr3, b3, g3, be3):
    n = x.shape[0]
    e = edge_index.shape[1]
    c_chunks = 8 * -(-e // (NW * CHUNK * 8))
    e_pad = NW * c_chunks * CHUNK
    r = -(-(n + 1) // (NS * 8)) * 8
    np_rows = NS * r

    ei = edge_index.astype(jnp.int32)
    src = jnp.concatenate([ei[0], jnp.zeros((e_pad - e,), jnp.int32)])
    dst = jnp.concatenate([ei[1], jnp.full((e_pad - e,), n, jnp.int32)])
    srcb = src.reshape(NW, c_chunks, CHUNK)
    dstb = dst.reshape(NW, c_chunks, CHUNK)

    d1 = Wl1.shape[0]
    d23 = Wl2.shape[0]
    z1 = jnp.zeros((np_rows, d1), jnp.float32)
    z23 = jnp.zeros((np_rows, d23), jnp.float32)
    z16 = jnp.zeros((np_rows, 16), jnp.float32)
    ones16 = jnp.ones((CHUNK, 16), jnp.float32)

    agg1 = _make_agg(d1, np_rows, c_chunks)
    agg23 = _make_agg(d23, np_rows, c_chunks)
    cntk = _make_cnt(np_rows, c_chunks)

    cntp = cntk(dstb, z16, ones16)
    y1 = _pre(x, Wl1)
    p1 = agg1(y1, srcb, dstb, z1)
    h1, y2, rinv = _mid(x, p1, cntp, Wr1, b1, g1, be1, Wl2, True)
    p2 = agg23(y2, srcb, dstb, z23)
    h2, y3 = _mid(h1, p2, rinv, Wr2, b2, g2, be2, Wl3, False)
    p3 = agg23(y3, srcb, dstb, z23)
    (h3,) = _mid(h2, p3, rinv, Wr3, b3, g3, be3, None, False)
    return h3


# speedup vs baseline: 1.4376x; 1.3580x over previous
import functools

import jax
import jax.numpy as jnp
from jax import lax
from jax.experimental import pallas as pl
from jax.experimental.pallas import tpu as pltpu
from jax.experimental.pallas import tpu_sc as plsc

NC = 2
NS = 16
NW = NC * NS
CHUNK = 128
BLK = 400


def _ln(z, g, b, eps=1e-5):
    mu = jnp.mean(z, axis=1, keepdims=True)
    d = z - mu
    var = jnp.mean(d * d, axis=1, keepdims=True)
    return d * lax.rsqrt(var + eps) * g + b


def _dot_t(a, w):
    return lax.dot_general(a, w, (((1,), (1,)), ((), ())),
                           preferred_element_type=jnp.float32)


def _make_agg(d, np_rows, c_chunks, stage):
    r = np_rows // NS
    nbuf = 2
    assert c_chunks % nbuf == 0
    mesh = plsc.VectorSubcoreMesh(core_axis_name="c", subcore_axis_name="s")

    @functools.partial(
        pl.kernel,
        mesh=mesh,
        out_type=jax.ShapeDtypeStruct((NC, np_rows, d), jnp.float32),
        scratch_types=(
            [pltpu.VMEM((c_chunks, CHUNK), jnp.int32)] * 2
            + [pltpu.VMEM((CHUNK, d), jnp.float32)] * nbuf
            + [pltpu.VMEM_SHARED((np_rows, d), jnp.float32)] * (2 if stage
                                                                else 1)
            + [pltpu.SemaphoreType.DMA] * nbuf
        ),
        compiler_params=pltpu.CompilerParams(use_tc_tiling_on_sc=False),
    )
    def agg(y_hbm, srcb_hbm, dstb_hbm, zeros_hbm, out_hbm,
            src_v, dst_v, *rest):
        bufs = rest[:nbuf]
        acc = rest[nbuf]
        if stage:
            ysh = rest[nbuf + 1]
            sems = rest[nbuf + 2:]
        else:
            ysh = y_hbm
            sems = rest[nbuf + 1:]
        cid = lax.axis_index("c")
        sid = lax.axis_index("s")
        wid = sid * NC + cid
        pltpu.sync_copy(srcb_hbm.at[wid], src_v)
        pltpu.sync_copy(dstb_hbm.at[wid], dst_v)
        r0 = sid * r
        pltpu.sync_copy(zeros_hbm.at[pl.ds(r0, r)], acc.at[pl.ds(r0, r)])
        if stage:
            pltpu.sync_copy(y_hbm.at[pl.ds(r0, r)], ysh.at[pl.ds(r0, r)])
        plsc.subcore_barrier()

        last = c_chunks - 1
        for k in range(nbuf):
            pltpu.async_copy(ysh.at[src_v.at[k]], bufs[k], sems[k])

        def body(i, carry):
            j = nbuf * i
            for k in range(nbuf):
                pltpu.make_async_copy(
                    ysh.at[src_v.at[0]], bufs[k], sems[k]).wait()
                pltpu.sync_copy(bufs[k], acc.at[dst_v.at[j + k]], add=True)
                nxt = jnp.minimum(j + nbuf + k, last)
                pltpu.async_copy(ysh.at[src_v.at[nxt]], bufs[k], sems[k])
            return carry

        lax.fori_loop(0, c_chunks // nbuf, body, 0)
        for k in range(nbuf):
            pltpu.make_async_copy(
                ysh.at[src_v.at[0]], bufs[k], sems[k]).wait()
        plsc.subcore_barrier()
        pltpu.sync_copy(acc.at[pl.ds(r0, r)], out_hbm.at[cid, pl.ds(r0, r)])

    return agg


def _make_cnt(np_rows, c_chunks):
    r = np_rows // NS
    mesh = plsc.VectorSubcoreMesh(core_axis_name="c", subcore_axis_name="s")

    @functools.partial(
        pl.kernel,
        mesh=mesh,
        out_type=jax.ShapeDtypeStruct((NC, np_rows, 16), jnp.float32),
        scratch_types=[
            pltpu.VMEM((c_chunks, CHUNK), jnp.int32),
            pltpu.VMEM((CHUNK, 16), jnp.float32),
            pltpu.VMEM_SHARED((np_rows, 16), jnp.float32),
        ],
        compiler_params=pltpu.CompilerParams(use_tc_tiling_on_sc=False),
    )
    def cnt(dstb_hbm, zeros_hbm, ones_hbm, out_hbm, dst_v, ones_v, acc):
        cid = lax.axis_index("c")
        sid = lax.axis_index("s")
        wid = sid * NC + cid
        pltpu.sync_copy(dstb_hbm.at[wid], dst_v)
        pltpu.sync_copy(ones_hbm, ones_v)
        r0 = sid * r
        pltpu.sync_copy(zeros_hbm.at[pl.ds(r0, r)], acc.at[pl.ds(r0, r)])
        plsc.subcore_barrier()

        def body(j, carry):
            pltpu.sync_copy(ones_v, acc.at[dst_v.at[j]], add=True)
            return carry

        lax.fori_loop(0, c_chunks, body, 0)
        plsc.subcore_barrier()
        pltpu.sync_copy(acc.at[pl.ds(r0, r)], out_hbm.at[cid, pl.ds(r0, r)])

    return cnt


def _pre(x, wl1):
    n = x.shape[0]
    grid = (n // BLK,)

    def body(x_ref, w_ref, o_ref):
        o_ref[...] = _dot_t(x_ref[...], w_ref[...])

    return pl.pallas_call(
        body,
        grid=grid,
        in_specs=[
            pl.BlockSpec((BLK, x.shape[1]), lambda i: (i, 0)),
            pl.BlockSpec(wl1.shape, lambda i: (0, 0)),
        ],
        out_specs=pl.BlockSpec((BLK, wl1.shape[0]), lambda i: (i, 0)),
        out_shape=jax.ShapeDtypeStruct((n, wl1.shape[0]), jnp.float32),
    )(x, wl1)


def _mid(h_in, p, cr, wr, b, g, be, wl_next, first):
    n = h_in.shape[0]
    d_out = wr.shape[0]
    np_rows = p.shape[1]
    d_agg = p.shape[2]
    grid = (n // BLK,)

    def body2(*refs):
        i = 0
        h_ref = refs[i]; i += 1
        p_ref = refs[i]; i += 1
        cr_ref = refs[i]; i += 1
        wr_ref = refs[i]; i += 1
        b_ref = refs[i]; i += 1
        g_ref = refs[i]; i += 1
        be_ref = refs[i]; i += 1
        if wl_next is not None:
            wln_ref = refs[i]; i += 1
        h_out = refs[i]; i += 1
        if wl_next is not None:
            y_out = refs[i]; i += 1
        if first:
            rinv_out = refs[i]; i += 1

        s = p_ref[0] + p_ref[1]
        if first:
            cnt = cr_ref[0, :, 0:1] + cr_ref[1, :, 0:1]
            ri = 1.0 / jnp.maximum(cnt, 1.0)
        else:
            ri = cr_ref[:, 0:1]
        mean = s * ri
        z = mean + b_ref[...] + _dot_t(h_ref[...], wr_ref[...])
        h = _ln(jnp.maximum(z, 0.0), g_ref[...], be_ref[...])
        h_out[...] = h
        if wl_next is not None:
            y_out[...] = _dot_t(h, wln_ref[...])
        if first:
            rinv_out[...] = jnp.broadcast_to(ri, (BLK, 16))

    in_specs = [
        pl.BlockSpec((BLK, h_in.shape[1]), lambda i: (i, 0)),
        pl.BlockSpec((NC, BLK, d_agg), lambda i: (0, i, 0)),
    ]
    operands = [h_in, p]
    if first:
        in_specs.append(pl.BlockSpec((NC, BLK, 16), lambda i: (0, i, 0)))
    else:
        in_specs.append(pl.BlockSpec((BLK, 16), lambda i: (i, 0)))
    operands.append(cr)
    in_specs += [
        pl.BlockSpec(wr.shape, lambda i: (0, 0)),
        pl.BlockSpec((1, d_out), lambda i: (0, 0)),
        pl.BlockSpec((1, d_out), lambda i: (0, 0)),
        pl.BlockSpec((1, d_out), lambda i: (0, 0)),
    ]
    operands += [wr, b.reshape(1, -1), g.reshape(1, -1), be.reshape(1, -1)]
    if wl_next is not None:
        in_specs.append(pl.BlockSpec(wl_next.shape, lambda i: (0, 0)))
        operands.append(wl_next)

    out_specs = [pl.BlockSpec((BLK, d_out), lambda i: (i, 0))]
    out_shape = [jax.ShapeDtypeStruct((n, d_out), jnp.float32)]
    if wl_next is not None:
        out_specs.append(pl.BlockSpec((BLK, wl_next.shape[0]), lambda i: (i, 0)))
        out_shape.append(jax.ShapeDtypeStruct((np_rows, wl_next.shape[0]),
                                              jnp.float32))
    if first:
        out_specs.append(pl.BlockSpec((BLK, 16), lambda i: (i, 0)))
        out_shape.append(jax.ShapeDtypeStruct((n, 16), jnp.float32))

    return pl.pallas_call(
        body2,
        grid=grid,
        in_specs=in_specs,
        out_specs=out_specs,
        out_shape=out_shape,
    )(*operands)


def kernel(x, edge_index, Wl1, Wr1, b1, g1, be1,
           Wl2, Wr2, b2, g2, be2, Wl3, Wr3, b3, g3, be3):
    n = x.shape[0]
    e = edge_index.shape[1]
    c_chunks = 8 * -(-e // (NW * CHUNK * 8))
    e_pad = NW * c_chunks * CHUNK
    r = -(-(n + 1) // (NS * 8)) * 8
    np_rows = NS * r

    ei = edge_index.astype(jnp.int32)
    src = jnp.concatenate([ei[0], jnp.zeros((e_pad - e,), jnp.int32)])
    dst = jnp.concatenate([ei[1], jnp.full((e_pad - e,), n, jnp.int32)])
    srcb = src.reshape(NW, c_chunks, CHUNK)
    dstb = dst.reshape(NW, c_chunks, CHUNK)

    d1 = Wl1.shape[0]
    d23 = Wl2.shape[0]
    z1 = jnp.zeros((np_rows, d1), jnp.float32)
    z23 = jnp.zeros((np_rows, d23), jnp.float32)
    z16 = jnp.zeros((np_rows, 16), jnp.float32)
    ones16 = jnp.ones((CHUNK, 16), jnp.float32)

    agg1 = _make_agg(d1, np_rows, c_chunks, False)
    agg23 = _make_agg(d23, np_rows, c_chunks, True)
    cntk = _make_cnt(np_rows, c_chunks)

    cntp = cntk(dstb, z16, ones16)
    y1 = _pre(x, Wl1)
    p1 = agg1(y1, srcb, dstb, z1)
    h1, y2, rinv = _mid(x, p1, cntp, Wr1, b1, g1, be1, Wl2, True)
    p2 = agg23(y2, srcb, dstb, z23)
    h2, y3 = _mid(h1, p2, rinv, Wr2, b2, g2, be2, Wl3, False)
    p3 = agg23(y3, srcb, dstb, z23)
    (h3,) = _mid(h2, p3, rinv, Wr3, b3, g3, be3, None, False)
    return h3


# speedup vs baseline: 2.0089x; 1.3974x over previous
import functools

import jax
import jax.numpy as jnp
from jax import lax
from jax.experimental import pallas as pl
from jax.experimental.pallas import tpu as pltpu
from jax.experimental.pallas import tpu_sc as plsc

NC = 2
NS = 16
NW = NC * NS
CHUNK = 128
BLK = 400


def _ln(z, g, b, eps=1e-5):
    mu = jnp.mean(z, axis=1, keepdims=True)
    d = z - mu
    var = jnp.mean(d * d, axis=1, keepdims=True)
    return d * lax.rsqrt(var + eps) * g + b


def _dot_t(a, w):
    return lax.dot_general(a, w, (((1,), (1,)), ((), ())),
                           preferred_element_type=jnp.float32)


def _make_agg(d, np_rows, c_chunks, stage):
    r = np_rows // NS
    nbuf = 2
    assert c_chunks % nbuf == 0
    mesh = plsc.VectorSubcoreMesh(core_axis_name="c", subcore_axis_name="s")

    @functools.partial(
        pl.kernel,
        mesh=mesh,
        out_type=jax.ShapeDtypeStruct((NC, np_rows, d), jnp.float32),
        scratch_types=(
            [pltpu.VMEM((c_chunks, CHUNK), jnp.int32)] * 2
            + [pltpu.VMEM((CHUNK, d), jnp.float32)] * nbuf
            + [pltpu.VMEM_SHARED((np_rows, d), jnp.float32)] * (2 if stage
                                                                else 1)
            + [pltpu.SemaphoreType.DMA] * nbuf
        ),
        compiler_params=pltpu.CompilerParams(use_tc_tiling_on_sc=False),
    )
    def agg(y_hbm, srcb_hbm, dstb_hbm, zeros_hbm, out_hbm,
            src_v, dst_v, *rest):
        bufs = rest[:nbuf]
        acc = rest[nbuf]
        if stage:
            ysh = rest[nbuf + 1]
            sems = rest[nbuf + 2:]
        else:
            ysh = y_hbm
            sems = rest[nbuf + 1:]
        cid = lax.axis_index("c")
        sid = lax.axis_index("s")
        wid = sid * NC + cid
        pltpu.sync_copy(srcb_hbm.at[wid], src_v)
        pltpu.sync_copy(dstb_hbm.at[wid], dst_v)
        r0 = sid * r
        pltpu.sync_copy(zeros_hbm.at[pl.ds(r0, r)], acc.at[pl.ds(r0, r)])
        if stage:
            pltpu.sync_copy(y_hbm.at[pl.ds(r0, r)], ysh.at[pl.ds(r0, r)])
        plsc.subcore_barrier()

        last = c_chunks - 1
        for k in range(nbuf):
            pltpu.async_copy(ysh.at[src_v.at[k]], bufs[k], sems[k])

        def body(i, carry):
            j = nbuf * i
            for k in range(nbuf):
                pltpu.make_async_copy(
                    ysh.at[src_v.at[0]], bufs[k], sems[k]).wait()
                pltpu.sync_copy(bufs[k], acc.at[dst_v.at[j + k]], add=True)
                nxt = jnp.minimum(j + nbuf + k, last)
                pltpu.async_copy(ysh.at[src_v.at[nxt]], bufs[k], sems[k])
            return carry

        lax.fori_loop(0, c_chunks // nbuf, body, 0)
        for k in range(nbuf):
            pltpu.make_async_copy(
                ysh.at[src_v.at[0]], bufs[k], sems[k]).wait()
        plsc.subcore_barrier()
        pltpu.sync_copy(acc.at[pl.ds(r0, r)], out_hbm.at[cid, pl.ds(r0, r)])

    return agg


def _make_cnt(np_rows, c_chunks):
    r = np_rows // NS
    mesh = plsc.VectorSubcoreMesh(core_axis_name="c", subcore_axis_name="s")

    @functools.partial(
        pl.kernel,
        mesh=mesh,
        out_type=jax.ShapeDtypeStruct((NC, np_rows, 16), jnp.float32),
        scratch_types=[
            pltpu.VMEM((c_chunks, CHUNK), jnp.int32),
            pltpu.VMEM((CHUNK, 16), jnp.float32),
            pltpu.VMEM_SHARED((np_rows, 16), jnp.float32),
        ],
        compiler_params=pltpu.CompilerParams(use_tc_tiling_on_sc=False),
    )
    def cnt(dstb_hbm, zeros_hbm, ones_hbm, out_hbm, dst_v, ones_v, acc):
        cid = lax.axis_index("c")
        sid = lax.axis_index("s")
        wid = sid * NC + cid
        pltpu.sync_copy(dstb_hbm.at[wid], dst_v)
        pltpu.sync_copy(ones_hbm, ones_v)
        r0 = sid * r
        pltpu.sync_copy(zeros_hbm.at[pl.ds(r0, r)], acc.at[pl.ds(r0, r)])
        plsc.subcore_barrier()

        def body(j, carry):
            pltpu.sync_copy(ones_v, acc.at[dst_v.at[j]], add=True)
            return carry

        lax.fori_loop(0, c_chunks, body, 0)
        plsc.subcore_barrier()
        pltpu.sync_copy(acc.at[pl.ds(r0, r)], out_hbm.at[cid, pl.ds(r0, r)])

    return cnt


def _pre(x, wl1, np_rows):
    n = x.shape[0]
    dh = wl1.shape[0] // 2
    grid = (n // BLK,)

    def body(x_ref, w_ref, oa_ref, ob_ref):
        y = _dot_t(x_ref[...], w_ref[...])
        oa_ref[...] = y[:, :dh]
        ob_ref[...] = y[:, dh:]

    return pl.pallas_call(
        body,
        grid=grid,
        in_specs=[
            pl.BlockSpec((BLK, x.shape[1]), lambda i: (i, 0)),
            pl.BlockSpec(wl1.shape, lambda i: (0, 0)),
        ],
        out_specs=[pl.BlockSpec((BLK, dh), lambda i: (i, 0))] * 2,
        out_shape=[jax.ShapeDtypeStruct((np_rows, dh), jnp.float32)] * 2,
    )(x, wl1)


def _mid(h_in, ps, cr, wr, b, g, be, wl_next, first):
    n = h_in.shape[0]
    d_out = wr.shape[0]
    np_rows = ps[0].shape[1]
    grid = (n // BLK,)

    def body2(*refs):
        i = 0
        h_ref = refs[i]; i += 1
        p_refs = refs[i:i + len(ps)]; i += len(ps)
        cr_ref = refs[i]; i += 1
        wr_ref = refs[i]; i += 1
        b_ref = refs[i]; i += 1
        g_ref = refs[i]; i += 1
        be_ref = refs[i]; i += 1
        if wl_next is not None:
            wln_ref = refs[i]; i += 1
        h_out = refs[i]; i += 1
        if wl_next is not None:
            y_out = refs[i]; i += 1
        if first:
            rinv_out = refs[i]; i += 1

        parts = [pr[0] + pr[1] for pr in p_refs]
        s = parts[0] if len(parts) == 1 else jnp.concatenate(parts, axis=1)
        if first:
            cnt = cr_ref[0, :, 0:1] + cr_ref[1, :, 0:1]
            ri = 1.0 / jnp.maximum(cnt, 1.0)
        else:
            ri = cr_ref[:, 0:1]
        mean = s * ri
        z = mean + b_ref[...] + _dot_t(h_ref[...], wr_ref[...])
        h = _ln(jnp.maximum(z, 0.0), g_ref[...], be_ref[...])
        h_out[...] = h
        if wl_next is not None:
            y_out[...] = _dot_t(h, wln_ref[...])
        if first:
            rinv_out[...] = jnp.broadcast_to(ri, (BLK, 16))

    in_specs = [pl.BlockSpec((BLK, h_in.shape[1]), lambda i: (i, 0))]
    in_specs += [pl.BlockSpec((NC, BLK, pp.shape[2]), lambda i: (0, i, 0))
                 for pp in ps]
    operands = [h_in] + list(ps)
    if first:
        in_specs.append(pl.BlockSpec((NC, BLK, 16), lambda i: (0, i, 0)))
    else:
        in_specs.append(pl.BlockSpec((BLK, 16), lambda i: (i, 0)))
    operands.append(cr)
    in_specs += [
        pl.BlockSpec(wr.shape, lambda i: (0, 0)),
        pl.BlockSpec((1, d_out), lambda i: (0, 0)),
        pl.BlockSpec((1, d_out), lambda i: (0, 0)),
        pl.BlockSpec((1, d_out), lambda i: (0, 0)),
    ]
    operands += [wr, b.reshape(1, -1), g.reshape(1, -1), be.reshape(1, -1)]
    if wl_next is not None:
        in_specs.append(pl.BlockSpec(wl_next.shape, lambda i: (0, 0)))
        operands.append(wl_next)

    out_specs = [pl.BlockSpec((BLK, d_out), lambda i: (i, 0))]
    out_shape = [jax.ShapeDtypeStruct((n, d_out), jnp.float32)]
    if wl_next is not None:
        out_specs.append(pl.BlockSpec((BLK, wl_next.shape[0]), lambda i: (i, 0)))
        out_shape.append(jax.ShapeDtypeStruct((np_rows, wl_next.shape[0]),
                                              jnp.float32))
    if first:
        out_specs.append(pl.BlockSpec((BLK, 16), lambda i: (i, 0)))
        out_shape.append(jax.ShapeDtypeStruct((n, 16), jnp.float32))

    return pl.pallas_call(
        body2,
        grid=grid,
        in_specs=in_specs,
        out_specs=out_specs,
        out_shape=out_shape,
    )(*operands)


def kernel(x, edge_index, Wl1, Wr1, b1, g1, be1,
           Wl2, Wr2, b2, g2, be2, Wl3, Wr3, b3, g3, be3):
    n = x.shape[0]
    e = edge_index.shape[1]
    c_chunks = 8 * -(-e // (NW * CHUNK * 8))
    e_pad = NW * c_chunks * CHUNK
    r = -(-(n + 1) // (NS * 8)) * 8
    np_rows = NS * r

    ei = edge_index.astype(jnp.int32)
    src = jnp.concatenate([ei[0], jnp.zeros((e_pad - e,), jnp.int32)])
    dst = jnp.concatenate([ei[1], jnp.full((e_pad - e,), n, jnp.int32)])
    srcb = src.reshape(NW, c_chunks, CHUNK)
    dstb = dst.reshape(NW, c_chunks, CHUNK)

    d23 = Wl2.shape[0]
    z23 = jnp.zeros((np_rows, d23), jnp.float32)
    z16 = jnp.zeros((np_rows, 16), jnp.float32)
    ones16 = jnp.ones((CHUNK, 16), jnp.float32)

    agg64 = _make_agg(d23, np_rows, c_chunks, True)
    cntk = _make_cnt(np_rows, c_chunks)

    cntp = cntk(dstb, z16, ones16)
    y1a, y1b = _pre(x, Wl1, np_rows)
    p1a = agg64(y1a, srcb, dstb, z23)
    p1b = agg64(y1b, srcb, dstb, z23)
    h1, y2, rinv = _mid(x, [p1a, p1b], cntp, Wr1, b1, g1, be1, Wl2, True)
    p2 = agg64(y2, srcb, dstb, z23)
    h2, y3 = _mid(h1, [p2], rinv, Wr2, b2, g2, be2, Wl3, False)
    p3 = agg64(y3, srcb, dstb, z23)
    (h3,) = _mid(h2, [p3], rinv, Wr3, b3, g3, be3, None, False)
    return h3
